# baseline (speedup 1.0000x reference)
import functools

import jax
import jax.numpy as jnp
from jax import lax
from jax.experimental import pallas as pl
from jax.experimental.pallas import tpu as pltpu
from jax.experimental.pallas import tpu_sc as plsc

N_NODES = 10000
N_EDGES = 320000
D_EDGE = 16
N_GRAPHS = 128
EPS_GEN = 1e-7
BN_EPS = 1e-5

NC = 2
NS = 16
LANES = 16

NP = 10240
EB = 128
NBLK = N_EDGES // EB
NBF = NBLK // NS + 1
PKC = 80
PKPAD = 2504
RPT = NP // NS
DR = 80
ND = RPT // DR


@functools.cache
def _make_sc_aggr(F):
    FH = F // 2
    CH = FH // LANES
    FH2 = FH // 2
    RB = EB * FH2 // 128
    GR = FH2 // LANES
    EPR = 128 // FH2

    mesh = plsc.VectorSubcoreMesh(core_axis_name="c", subcore_axis_name="s")

    @functools.partial(
        pl.kernel,
        mesh=mesh,
        compiler_params=pltpu.CompilerParams(use_tc_tiling_on_sc=False,
                                             needs_layout_passes=False),
        out_type=jax.ShapeDtypeStruct((NC, NP, FH), jnp.float32),
        scratch_types=[
            pltpu.VMEM_SHARED((NP, FH), jnp.float32),
            pltpu.VMEM_SHARED((NP, FH), jnp.float32),
            pltpu.VMEM((PKC, EB), jnp.int32),
            pltpu.VMEM((EB,), jnp.int32),
            pltpu.VMEM((EB,), jnp.int32),
            pltpu.VMEM((EB,), jnp.int32),
            pltpu.VMEM((EB,), jnp.int32),
            pltpu.VMEM((EB, FH2), jnp.int32),
            pltpu.VMEM((EB, FH2), jnp.int32),
            pltpu.VMEM((EB, FH), jnp.float32),
            pltpu.VMEM((EB, FH), jnp.float32),
            pltpu.VMEM((EB, FH), jnp.float32),
            pltpu.VMEM((DR, FH), jnp.float32),
            pltpu.SemaphoreType.DMA,
            pltpu.SemaphoreType.DMA,
            pltpu.SemaphoreType.DMA,
            pltpu.SemaphoreType.DMA,
            pltpu.SemaphoreType.DMA,
        ],
    )
    def aggr(xs_hbm, pk_hbm, e_hbm, out_hbm,
             s_acc, t_acc, pkbuf, sv0, sv1, dv0, dv1,
             eb0, eb1, gb0, gb1, pbuf, dbt,
             lds0, lds1, scs0, scs1, psem):
        c = lax.axis_index("c")
        s = lax.axis_index("s")
        row0 = s * RPT
        sv, dv = [sv0, sv1], [dv0, dv1]
        eb, gb = [eb0, eb1], [gb0, gb1]
        lds, scs = [lds0, lds1], [scs0, scs1]
        blk0 = s * (NBF - 1) + jnp.minimum(s, NBLK % NS)
        nblk = jnp.where(s < NBLK % NS, NBF, NBF - 1)

        zero = jnp.zeros((LANES,), jnp.float32)

        def zrow(i, _):
            for k in range(CH):
                gb0[i, pl.ds(k * LANES, LANES)] = zero
            return 0

        lax.fori_loop(0, DR, zrow, 0)

        def zissue(d, _):
            r0 = row0 + d * DR
            pltpu.async_copy(gb0.at[pl.ds(0, DR)], s_acc.at[pl.ds(r0, DR)],
                             scs0)
            pltpu.async_copy(gb0.at[pl.ds(0, DR)], t_acc.at[pl.ds(r0, DR)],
                             scs0)
            return 0

        lax.fori_loop(0, ND, zissue, 0)
        pltpu.sync_copy(pk_hbm.at[pl.ds(blk0, PKC)], pkbuf)

        def zwait(d, _):
            r0 = row0 + d * DR
            pltpu.make_async_copy(gb0.at[pl.ds(0, DR)],
                                  s_acc.at[pl.ds(r0, DR)], scs0).wait()
            pltpu.make_async_copy(gb0.at[pl.ds(0, DR)],
                                  t_acc.at[pl.ds(r0, DR)], scs0).wait()
            return 0

        lax.fori_loop(0, ND, zwait, 0)
        plsc.subcore_barrier()

        def unpack(i, b):
            r = i % PKC
            for k in range(EB // LANES):
                sl = pl.ds(k * LANES, LANES)
                v = pkbuf[r, sl]
                sv[b][sl] = v & 0xFFFF
                dv[b][sl] = v >> 16

        def issue_loads(i, b):
            pltpu.async_copy(e_hbm.at[c, pl.ds((blk0 + i) * EB, EB)],
                             eb[b], lds[b])
            pltpu.async_copy(xs_hbm.at[c].at[sv[b]], gb[b], lds[b])

        unpack(0, 0)
        issue_loads(0, 0)

        def do_block(i, b):
            nb2 = 1 - b
            pltpu.make_async_copy(
                e_hbm.at[c, pl.ds((blk0 + i) * EB, EB)], eb[b],
                lds[b]).wait()
            pltpu.make_async_copy(
                xs_hbm.at[c].at[sv[b]], gb[b], lds[b]).wait()

            @pl.when(i == PKC - 1)
            def _():
                pltpu.sync_copy(pk_hbm.at[pl.ds(blk0 + PKC, PKC)], pkbuf)

            @pl.when(i >= 1)
            def _():
                pltpu.make_async_copy(pbuf, s_acc.at[dv[nb2]], psem).wait()

            @pl.when(i + 1 < nblk)
            def _():
                @pl.when(i >= 1)
                def _():
                    pltpu.make_async_copy(
                        gb[nb2], t_acc.at[dv[nb2]], scs[nb2]).wait()

                unpack(i + 1, nb2)
                issue_loads(i + 1, nb2)

            def crow(r, _):
                for g in range(GR):
                    v = eb[b][r, pl.ds(g * LANES, LANES)]
                    fa = plsc.bitcast(v << 16, jnp.float32)
                    fb = plsc.bitcast(v & jnp.int32(-65536), jnp.float32)
                    for f, col in ((fa, g * LANES), (fb, FH2 + g * LANES)):
                        sl = pl.ds(col, LANES)
                        msg = jnp.maximum(gb[b][r, sl] + f, 0.0) + EPS_GEN
                        p = jnp.exp(msg)
                        pbuf[r, sl] = p
                        gb[b][r, sl] = p * msg
                return 0

            lax.fori_loop(0, EB, crow, 0)
            pltpu.async_copy(pbuf, s_acc.at[dv[b]], psem, add=True)
            pltpu.async_copy(gb[b], t_acc.at[dv[b]], scs[b], add=True)

        def blockstep(i, _):
            @pl.when(i % 2 == 0)
            def _():
                do_block(i, 0)

            @pl.when(i % 2 == 1)
            def _():
                do_block(i, 1)

            return 0

        lax.fori_loop(0, nblk, blockstep, 0)
        pltpu.make_async_copy(pbuf, s_acc.at[dv0], psem).wait()
        for b in range(2):
            pltpu.make_async_copy(gb[b], t_acc.at[dv[b]], scs[b]).wait()
        plsc.subcore_barrier()

        dsb = [gb0, pbuf]
        dtb = [gb1, dbt]

        def dissue(d, b):
            r0 = row0 + d * DR
            pltpu.async_copy(s_acc.at[pl.ds(r0, DR)],
                             dsb[b].at[pl.ds(0, DR)], lds[b])
            pltpu.async_copy(t_acc.at[pl.ds(r0, DR)],
                             dtb[b].at[pl.ds(0, DR)], lds[b])

        dissue(0, 0)

        def dpair(dp, _):
            for b in range(2):
                d = dp * 2 + b
                r0 = row0 + d * DR
                pltpu.make_async_copy(
                    s_acc.at[pl.ds(r0, DR)], dsb[b].at[pl.ds(0, DR)],
                    lds[b]).wait()
                pltpu.make_async_copy(
                    t_acc.at[pl.ds(r0, DR)], dtb[b].at[pl.ds(0, DR)],
                    lds[b]).wait()

                @pl.when(d + 1 < ND)
                def _():
                    dissue(d + 1, 1 - b)

                def drow(r, _):
                    for k in range(CH):
                        sl = pl.ds(k * LANES, LANES)
                        dsb[b][r, sl] = dtb[b][r, sl] / (dsb[b][r, sl]
                                                         + 1e-16)
                    return 0

                lax.fori_loop(0, DR, drow, 0)
                pltpu.sync_copy(dsb[b].at[pl.ds(0, DR)],
                                out_hbm.at[c, pl.ds(r0, DR)])
            return 0

        lax.fori_loop(0, ND // 2, dpair, 0)

    return aggr


_EB_TC = 1600


def _pack_bf16_pairs(h):
    f = h.shape[1]
    fh = f // 2
    outs = []
    for c in range(2):
        half = h[:, c * fh:(c + 1) * fh]
        a = half[:, :fh // 2].astype(jnp.bfloat16).astype(jnp.float32)
        bcols = half[:, fh // 2:].astype(jnp.bfloat16).astype(jnp.float32)
        va = lax.bitcast_convert_type(a, jnp.int32)
        vb = lax.bitcast_convert_type(bcols, jnp.int32)
        outs.append(vb | lax.shift_right_logical(va, 16))
    return outs


def _edge_mlp_body(ea, w1, b1, w2, b2, w3, b3, o1, o2, o3):
    a = ea[...]
    h1 = jnp.dot(a, w1[...], preferred_element_type=jnp.float32) + b1[...]
    h2 = jnp.dot(a, w2[...], preferred_element_type=jnp.float32) + b2[...]
    h3 = jnp.dot(a, w3[...], preferred_element_type=jnp.float32) + b3[...]
    o1[0], o1[1] = _pack_bf16_pairs(h1)
    o2[0], o2[1] = _pack_bf16_pairs(h2)
    o3[0], o3[1] = _pack_bf16_pairs(h3)


def _edge_mlp(edge_attr, p1, p2, p3):
    nblk = N_EDGES // _EB_TC
    wspec = lambda shp: pl.BlockSpec(shp, lambda i: (0, 0))
    ospec = lambda fh2: pl.BlockSpec((NC, _EB_TC, fh2), lambda i: (0, i, 0))
    return pl.pallas_call(
        _edge_mlp_body,
        grid=(nblk,),
        in_specs=[
            pl.BlockSpec((_EB_TC, D_EDGE), lambda i: (i, 0)),
            wspec((D_EDGE, 128)), wspec((1, 128)),
            wspec((D_EDGE, 64)), wspec((1, 64)),
            wspec((D_EDGE, 128)), wspec((1, 128)),
        ],
        out_specs=[ospec(32), ospec(16), ospec(32)],
        out_shape=[
            jax.ShapeDtypeStruct((NC, N_EDGES, 32), jnp.int32),
            jax.ShapeDtypeStruct((NC, N_EDGES, 16), jnp.int32),
            jax.ShapeDtypeStruct((NC, N_EDGES, 32), jnp.int32),
        ],
    )(edge_attr,
      p1["We"], p1["be"].reshape(1, -1),
      p2["We"], p2["be"].reshape(1, -1),
      p3["We"], p3["be"].reshape(1, -1))


def _bn(h, g, b):
    mu = jnp.mean(h, axis=0, keepdims=True)
    var = jnp.mean((h - mu) ** 2, axis=0, keepdims=True)
    return (h - mu) * lax.rsqrt(var + BN_EPS) * g + b


def _node_body(o_ref, x_ref, wa, ba, gm, bm, wb, bb, gn, bnb, y_ref):
    out = jnp.concatenate([o_ref[0, :N_NODES] + x_ref[0, :N_NODES],
                           o_ref[1, :N_NODES] + x_ref[1, :N_NODES]], axis=1)
    h = jnp.dot(out, wa[...], preferred_element_type=jnp.float32) + ba[...]
    h = jnp.maximum(_bn(h, gm[...], bm[...]), 0.0)
    h = jnp.dot(h, wb[...], preferred_element_type=jnp.float32) + bb[...]
    y = jnp.maximum(_bn(h, gn[...], bnb[...]), 0.0)
    fh = y.shape[1] // 2
    y_ref[0], y_ref[1] = y[:, :fh], y[:, fh:]


def _node_mlp(aggr_out, x_in, p):
    dout = p["Wb"].shape[1]
    return pl.pallas_call(
        _node_body,
        out_shape=jax.ShapeDtypeStruct((NC, N_NODES, dout // 2), jnp.float32),
    )(aggr_out, x_in, p["Wa"], p["ba"].reshape(1, -1), p["gm"].reshape(1, -1),
      p["bm"].reshape(1, -1), p["Wb"], p["bb"].reshape(1, -1),
      p["gn"].reshape(1, -1), p["bn"].reshape(1, -1))


def _node_pool_body(o_ref, x_ref, wa, ba, gm, bm, wb, bb, gn, bnb, batch_ref,
                    out_ref):
    out = jnp.concatenate([o_ref[0, :N_NODES] + x_ref[0, :N_NODES],
                           o_ref[1, :N_NODES] + x_ref[1, :N_NODES]], axis=1)
    h = jnp.dot(out, wa[...], preferred_element_type=jnp.float32) + ba[...]
    h = jnp.maximum(_bn(h, gm[...], bm[...]), 0.0)
    h = jnp.dot(h, wb[...], preferred_element_type=jnp.float32) + bb[...]
    y = jnp.maximum(_bn(h, gn[...], bnb[...]), 0.0)
    onehot = (batch_ref[...] == lax.broadcasted_iota(
        jnp.int32, (1, N_GRAPHS), 1)).astype(jnp.float32)
    s = lax.dot_general(onehot, y, (((0,), (0,)), ((), ())),
                        preferred_element_type=jnp.float32)
    cnt = jnp.sum(onehot, axis=0)
    out_ref[...] = s / jnp.maximum(cnt, 1.0)[:, None]


def _node_pool(aggr_out, x_in, batch2d, p):
    return pl.pallas_call(
        _node_pool_body,
        out_shape=jax.ShapeDtypeStruct((N_GRAPHS, p["Wb"].shape[1]),
                                       jnp.float32),
    )(aggr_out, x_in, p["Wa"], p["ba"].reshape(1, -1), p["gm"].reshape(1, -1),
      p["bm"].reshape(1, -1), p["Wb"], p["bb"].reshape(1, -1),
      p["gn"].reshape(1, -1), p["bn"].reshape(1, -1), batch2d)


def kernel(x, edge_index, edge_attr, batch, params):
    pk = (edge_index[0] | (edge_index[1] << 16)).reshape(NBLK, EB)
    pk = jnp.pad(pk, ((0, PKPAD - NBLK), (0, 0)))
    batch2d = batch.reshape(N_NODES, 1)

    e1, e2, e3 = _edge_mlp(edge_attr, params["l1"], params["l2"], params["l3"])

    sc128 = _make_sc_aggr(128)
    sc64 = _make_sc_aggr(64)
    pad = lambda a: jnp.pad(a, ((0, 0), (0, NP - N_NODES), (0, 0)))
    x0 = pad(x.reshape(N_NODES, NC, 64).transpose(1, 0, 2))
    a1 = sc128(x0, pk, e1)
    x1 = pad(_node_mlp(a1, x0, params["l1"]))
    a2 = sc64(x1, pk, e2)
    x2 = pad(_node_mlp(a2, x1, params["l2"]))
    a3 = sc128(x2, pk, e3)
    return _node_pool(a3, x2, batch2d, params["l3"])


# speedup vs baseline: 1.0006x; 1.0006x over previous
import functools

import jax
import jax.numpy as jnp
from jax import lax
from jax.experimental import pallas as pl
from jax.experimental.pallas import tpu as pltpu
from jax.experimental.pallas import tpu_sc as plsc

N_NODES = 10000
N_EDGES = 320000
D_EDGE = 16
N_GRAPHS = 128
EPS_GEN = 1e-7
BN_EPS = 1e-5

NC = 2
NS = 16
LANES = 16

NP = 10240
EB = 128
NBLK = N_EDGES // EB
NBF = NBLK // NS + 1
PKC = 40
PKPAD = 2504
RPT = NP // NS
DR = 80
ND = RPT // DR


@functools.cache
def _make_sc_aggr(F):
    FH = F // 2
    CH = FH // LANES
    FH2 = FH // 2
    GR = FH2 // LANES

    mesh = plsc.VectorSubcoreMesh(core_axis_name="c", subcore_axis_name="s")

    @functools.partial(
        pl.kernel,
        mesh=mesh,
        compiler_params=pltpu.CompilerParams(use_tc_tiling_on_sc=False,
                                             needs_layout_passes=False),
        out_type=jax.ShapeDtypeStruct((NC, NP, FH), jnp.float32),
        scratch_types=[
            pltpu.VMEM_SHARED((NP, FH), jnp.float32),
            pltpu.VMEM_SHARED((NP, FH), jnp.float32),
            pltpu.VMEM((PKC, EB), jnp.int32),
            pltpu.VMEM((EB,), jnp.int32),
            pltpu.VMEM((EB,), jnp.int32),
            pltpu.VMEM((EB,), jnp.int32),
            pltpu.VMEM((EB,), jnp.int32),
            pltpu.VMEM((EB, FH2), jnp.int32),
            pltpu.VMEM((EB, FH2), jnp.int32),
            pltpu.VMEM((EB, FH), jnp.float32),
            pltpu.VMEM((EB, FH), jnp.float32),
            pltpu.VMEM((EB, FH), jnp.float32),
            pltpu.VMEM((EB, FH), jnp.float32),
            pltpu.SemaphoreType.DMA,
            pltpu.SemaphoreType.DMA,
            pltpu.SemaphoreType.DMA,
            pltpu.SemaphoreType.DMA,
            pltpu.SemaphoreType.DMA,
            pltpu.SemaphoreType.DMA,
        ],
    )
    def aggr(xs_hbm, pk_hbm, e_hbm, out_hbm,
             s_acc, t_acc, pkbuf, sv0, sv1, dv0, dv1,
             eb0, eb1, gb0, gb1, pb0, pb1,
             lds0, lds1, scs0, scs1, psm0, psm1):
        c = lax.axis_index("c")
        s = lax.axis_index("s")
        row0 = s * RPT
        sv, dv = [sv0, sv1], [dv0, dv1]
        eb, gb, pb = [eb0, eb1], [gb0, gb1], [pb0, pb1]
        lds, scs, psm = [lds0, lds1], [scs0, scs1], [psm0, psm1]
        blk0 = s * (NBF - 1) + jnp.minimum(s, NBLK % NS)
        nblk = jnp.where(s < NBLK % NS, NBF, NBF - 1)

        zero = jnp.zeros((LANES,), jnp.float32)

        def zrow(i, _):
            for k in range(CH):
                gb0[i, pl.ds(k * LANES, LANES)] = zero
            return 0

        lax.fori_loop(0, DR, zrow, 0)

        def zissue(d, _):
            r0 = row0 + d * DR
            pltpu.async_copy(gb0.at[pl.ds(0, DR)], s_acc.at[pl.ds(r0, DR)],
                             scs0)
            pltpu.async_copy(gb0.at[pl.ds(0, DR)], t_acc.at[pl.ds(r0, DR)],
                             scs0)
            return 0

        lax.fori_loop(0, ND, zissue, 0)
        pltpu.sync_copy(pk_hbm.at[pl.ds(blk0, PKC)], pkbuf)

        def zwait(d, _):
            r0 = row0 + d * DR
            pltpu.make_async_copy(gb0.at[pl.ds(0, DR)],
                                  s_acc.at[pl.ds(r0, DR)], scs0).wait()
            pltpu.make_async_copy(gb0.at[pl.ds(0, DR)],
                                  t_acc.at[pl.ds(r0, DR)], scs0).wait()
            return 0

        lax.fori_loop(0, ND, zwait, 0)
        plsc.subcore_barrier()

        def unpack(i, b):
            r = i % PKC
            for k in range(EB // LANES):
                sl = pl.ds(k * LANES, LANES)
                v = pkbuf[r, sl]
                sv[b][sl] = v & 0xFFFF
                dv[b][sl] = v >> 16

        def issue_loads(i, b):
            pltpu.async_copy(e_hbm.at[c, pl.ds((blk0 + i) * EB, EB)],
                             eb[b], lds[b])
            pltpu.async_copy(xs_hbm.at[c].at[sv[b]], gb[b], lds[b])

        unpack(0, 0)
        issue_loads(0, 0)

        def do_block(i, b):
            nb2 = 1 - b
            pltpu.make_async_copy(
                e_hbm.at[c, pl.ds((blk0 + i) * EB, EB)], eb[b],
                lds[b]).wait()
            pltpu.make_async_copy(
                xs_hbm.at[c].at[sv[b]], gb[b], lds[b]).wait()

            @pl.when(jnp.logical_and((i + 1) % PKC == 0, i + 1 < nblk))
            def _():
                pltpu.sync_copy(pk_hbm.at[pl.ds(blk0 + i + 1, PKC)], pkbuf)

            @pl.when(i + 1 < nblk)
            def _():
                @pl.when(i >= 1)
                def _():
                    pltpu.make_async_copy(
                        pb[nb2], s_acc.at[dv[nb2]], psm[nb2]).wait()
                    pltpu.make_async_copy(
                        gb[nb2], t_acc.at[dv[nb2]], scs[nb2]).wait()

                unpack(i + 1, nb2)
                issue_loads(i + 1, nb2)

            def crow(r, _):
                for g in range(GR):
                    v = eb[b][r, pl.ds(g * LANES, LANES)]
                    fa = plsc.bitcast(v << 16, jnp.float32)
                    fb = plsc.bitcast(v & jnp.int32(-65536), jnp.float32)
                    for f, col in ((fa, g * LANES), (fb, FH2 + g * LANES)):
                        sl = pl.ds(col, LANES)
                        msg = jnp.maximum(gb[b][r, sl] + f, 0.0) + EPS_GEN
                        p = jnp.exp(msg)
                        pb[b][r, sl] = p
                        gb[b][r, sl] = p * msg
                return 0

            lax.fori_loop(0, EB, crow, 0)
            pltpu.async_copy(pb[b], s_acc.at[dv[b]], psm[b], add=True)
            pltpu.async_copy(gb[b], t_acc.at[dv[b]], scs[b], add=True)

        def blockstep(i, _):
            @pl.when(i % 2 == 0)
            def _():
                do_block(i, 0)

            @pl.when(i % 2 == 1)
            def _():
                do_block(i, 1)

            return 0

        lax.fori_loop(0, nblk, blockstep, 0)
        for b in range(2):
            pltpu.make_async_copy(pb[b], s_acc.at[dv[b]], psm[b]).wait()
            pltpu.make_async_copy(gb[b], t_acc.at[dv[b]], scs[b]).wait()
        plsc.subcore_barrier()

        def dissue(d, b):
            r0 = row0 + d * DR
            pltpu.async_copy(s_acc.at[pl.ds(r0, DR)], pb[b].at[pl.ds(0, DR)],
                             lds[b])
            pltpu.async_copy(t_acc.at[pl.ds(r0, DR)], gb[b].at[pl.ds(0, DR)],
                             lds[b])

        dissue(0, 0)

        def dpair(dp, _):
            for b in range(2):
                d = dp * 2 + b
                r0 = row0 + d * DR
                pltpu.make_async_copy(
                    s_acc.at[pl.ds(r0, DR)], pb[b].at[pl.ds(0, DR)],
                    lds[b]).wait()
                pltpu.make_async_copy(
                    t_acc.at[pl.ds(r0, DR)], gb[b].at[pl.ds(0, DR)],
                    lds[b]).wait()

                @pl.when(d + 1 < ND)
                def _():
                    dissue(d + 1, 1 - b)

                def drow(r, _):
                    for k in range(CH):
                        sl = pl.ds(k * LANES, LANES)
                        pb[b][r, sl] = gb[b][r, sl] / (pb[b][r, sl] + 1e-16)
                    return 0

                lax.fori_loop(0, DR, drow, 0)
                pltpu.sync_copy(pb[b].at[pl.ds(0, DR)],
                                out_hbm.at[c, pl.ds(r0, DR)])
            return 0

        lax.fori_loop(0, ND // 2, dpair, 0)

    return aggr


_EB_TC = 1600


def _pack_bf16_pairs(h):
    f = h.shape[1]
    fh = f // 2
    outs = []
    for c in range(2):
        half = h[:, c * fh:(c + 1) * fh]
        a = half[:, :fh // 2].astype(jnp.bfloat16).astype(jnp.float32)
        bcols = half[:, fh // 2:].astype(jnp.bfloat16).astype(jnp.float32)
        va = lax.bitcast_convert_type(a, jnp.int32)
        vb = lax.bitcast_convert_type(bcols, jnp.int32)
        outs.append(vb | lax.shift_right_logical(va, 16))
    return outs


def _edge_mlp_body(ea, w1, b1, w2, b2, w3, b3, o1, o2, o3):
    a = ea[...]
    h1 = jnp.dot(a, w1[...], preferred_element_type=jnp.float32) + b1[...]
    h2 = jnp.dot(a, w2[...], preferred_element_type=jnp.float32) + b2[...]
    h3 = jnp.dot(a, w3[...], preferred_element_type=jnp.float32) + b3[...]
    o1[0], o1[1] = _pack_bf16_pairs(h1)
    o2[0], o2[1] = _pack_bf16_pairs(h2)
    o3[0], o3[1] = _pack_bf16_pairs(h3)


def _edge_mlp(edge_attr, p1, p2, p3):
    nblk = N_EDGES // _EB_TC
    wspec = lambda shp: pl.BlockSpec(shp, lambda i: (0, 0))
    ospec = lambda fh2: pl.BlockSpec((NC, _EB_TC, fh2), lambda i: (0, i, 0))
    return pl.pallas_call(
        _edge_mlp_body,
        grid=(nblk,),
        in_specs=[
            pl.BlockSpec((_EB_TC, D_EDGE), lambda i: (i, 0)),
            wspec((D_EDGE, 128)), wspec((1, 128)),
            wspec((D_EDGE, 64)), wspec((1, 64)),
            wspec((D_EDGE, 128)), wspec((1, 128)),
        ],
        out_specs=[ospec(32), ospec(16), ospec(32)],
        out_shape=[
            jax.ShapeDtypeStruct((NC, N_EDGES, 32), jnp.int32),
            jax.ShapeDtypeStruct((NC, N_EDGES, 16), jnp.int32),
            jax.ShapeDtypeStruct((NC, N_EDGES, 32), jnp.int32),
        ],
    )(edge_attr,
      p1["We"], p1["be"].reshape(1, -1),
      p2["We"], p2["be"].reshape(1, -1),
      p3["We"], p3["be"].reshape(1, -1))


def _bn(h, g, b):
    mu = jnp.mean(h, axis=0, keepdims=True)
    var = jnp.mean((h - mu) ** 2, axis=0, keepdims=True)
    return (h - mu) * lax.rsqrt(var + BN_EPS) * g + b


def _node_body(o_ref, x_ref, wa, ba, gm, bm, wb, bb, gn, bnb, y_ref):
    out = jnp.concatenate([o_ref[0, :N_NODES] + x_ref[0, :N_NODES],
                           o_ref[1, :N_NODES] + x_ref[1, :N_NODES]], axis=1)
    h = jnp.dot(out, wa[...], preferred_element_type=jnp.float32) + ba[...]
    h = jnp.maximum(_bn(h, gm[...], bm[...]), 0.0)
    h = jnp.dot(h, wb[...], preferred_element_type=jnp.float32) + bb[...]
    y = jnp.maximum(_bn(h, gn[...], bnb[...]), 0.0)
    fh = y.shape[1] // 2
    y_ref[0], y_ref[1] = y[:, :fh], y[:, fh:]


def _node_mlp(aggr_out, x_in, p):
    dout = p["Wb"].shape[1]
    return pl.pallas_call(
        _node_body,
        out_shape=jax.ShapeDtypeStruct((NC, N_NODES, dout // 2), jnp.float32),
    )(aggr_out, x_in, p["Wa"], p["ba"].reshape(1, -1), p["gm"].reshape(1, -1),
      p["bm"].reshape(1, -1), p["Wb"], p["bb"].reshape(1, -1),
      p["gn"].reshape(1, -1), p["bn"].reshape(1, -1))


def _node_pool_body(o_ref, x_ref, wa, ba, gm, bm, wb, bb, gn, bnb, batch_ref,
                    out_ref):
    out = jnp.concatenate([o_ref[0, :N_NODES] + x_ref[0, :N_NODES],
                           o_ref[1, :N_NODES] + x_ref[1, :N_NODES]], axis=1)
    h = jnp.dot(out, wa[...], preferred_element_type=jnp.float32) + ba[...]
    h = jnp.maximum(_bn(h, gm[...], bm[...]), 0.0)
    h = jnp.dot(h, wb[...], preferred_element_type=jnp.float32) + bb[...]
    y = jnp.maximum(_bn(h, gn[...], bnb[...]), 0.0)
    onehot = (batch_ref[...] == lax.broadcasted_iota(
        jnp.int32, (1, N_GRAPHS), 1)).astype(jnp.float32)
    s = lax.dot_general(onehot, y, (((0,), (0,)), ((), ())),
                        preferred_element_type=jnp.float32)
    cnt = jnp.sum(onehot, axis=0)
    out_ref[...] = s / jnp.maximum(cnt, 1.0)[:, None]


def _node_pool(aggr_out, x_in, batch2d, p):
    return pl.pallas_call(
        _node_pool_body,
        out_shape=jax.ShapeDtypeStruct((N_GRAPHS, p["Wb"].shape[1]),
                                       jnp.float32),
    )(aggr_out, x_in, p["Wa"], p["ba"].reshape(1, -1), p["gm"].reshape(1, -1),
      p["bm"].reshape(1, -1), p["Wb"], p["bb"].reshape(1, -1),
      p["gn"].reshape(1, -1), p["bn"].reshape(1, -1), batch2d)


def kernel(x, edge_index, edge_attr, batch, params):
    pk = (edge_index[0] | (edge_index[1] << 16)).reshape(NBLK, EB)
    pk = jnp.pad(pk, ((0, PKPAD - NBLK), (0, 0)))
    batch2d = batch.reshape(N_NODES, 1)

    e1, e2, e3 = _edge_mlp(edge_attr, params["l1"], params["l2"], params["l3"])

    sc128 = _make_sc_aggr(128)
    sc64 = _make_sc_aggr(64)
    pad = lambda a: jnp.pad(a, ((0, 0), (0, NP - N_NODES), (0, 0)))
    x0 = pad(x.reshape(N_NODES, NC, 64).transpose(1, 0, 2))
    a1 = sc128(x0, pk, e1)
    x1 = pad(_node_mlp(a1, x0, params["l1"]))
    a2 = sc64(x1, pk, e2)
    x2 = pad(_node_mlp(a2, x1, params["l2"]))
    a3 = sc128(x2, pk, e3)
    return _node_pool(a3, x2, batch2d, params["l3"])


# speedup vs baseline: 1.3609x; 1.3600x over previous
import functools

import jax
import jax.numpy as jnp
from jax import lax
from jax.experimental import pallas as pl
from jax.experimental.pallas import tpu as pltpu
from jax.experimental.pallas import tpu_sc as plsc

N_NODES = 10000
N_EDGES = 320000
D_EDGE = 16
N_GRAPHS = 128
EPS_GEN = 1e-7
BN_EPS = 1e-5

NC = 2
NS = 16
LANES = 16

NP = 10240
EB = 128
NBLK = N_EDGES // EB
NBF = NBLK // NS + 1
PKC = 80
PKPAD = 2504
RPT = NP // NS
DR = 80
ND = RPT // DR


@functools.cache
def _make_sc_aggr(F):
    FH = F // 2
    CH = FH // LANES

    mesh = plsc.VectorSubcoreMesh(core_axis_name="c", subcore_axis_name="s")

    @functools.partial(
        pl.kernel,
        mesh=mesh,
        compiler_params=pltpu.CompilerParams(use_tc_tiling_on_sc=False),
        out_type=jax.ShapeDtypeStruct((NC, NP, FH), jnp.float32),
        scratch_types=[
            pltpu.VMEM_SHARED((NP, FH), jnp.float32),
            pltpu.VMEM_SHARED((NP, FH), jnp.float32),
            pltpu.VMEM((PKC, EB), jnp.int32),
            pltpu.VMEM((EB,), jnp.int32),
            pltpu.VMEM((EB,), jnp.int32),
            pltpu.VMEM((EB,), jnp.int32),
            pltpu.VMEM((EB,), jnp.int32),
            pltpu.VMEM((EB, FH), jnp.float32),
            pltpu.VMEM((EB, FH), jnp.float32),
            pltpu.VMEM((EB, FH), jnp.float32),
            pltpu.VMEM((EB, FH), jnp.float32),
            pltpu.SemaphoreType.DMA,
            pltpu.SemaphoreType.DMA,
            pltpu.SemaphoreType.DMA,
            pltpu.SemaphoreType.DMA,
        ],
    )
    def aggr(xs_hbm, pk_hbm, e_hbm, out_hbm,
             s_acc, t_acc, pkbuf, sv0, sv1, dv0, dv1,
             eb0, eb1, gb0, gb1, lds0, lds1, scs0, scs1):
        c = lax.axis_index("c")
        s = lax.axis_index("s")
        row0 = s * RPT
        sv, dv = [sv0, sv1], [dv0, dv1]
        eb, gb = [eb0, eb1], [gb0, gb1]
        lds, scs = [lds0, lds1], [scs0, scs1]
        blk0 = s * (NBF - 1) + jnp.minimum(s, NBLK % NS)
        nblk = jnp.where(s < NBLK % NS, NBF, NBF - 1)

        zero = jnp.zeros((LANES,), jnp.float32)

        def zrow(i, _):
            for k in range(CH):
                eb0[i, pl.ds(k * LANES, LANES)] = zero
            return 0

        lax.fori_loop(0, DR, zrow, 0)

        def zissue(d, _):
            r0 = row0 + d * DR
            pltpu.async_copy(eb0.at[pl.ds(0, DR)], s_acc.at[pl.ds(r0, DR)],
                             scs0)
            pltpu.async_copy(eb0.at[pl.ds(0, DR)], t_acc.at[pl.ds(r0, DR)],
                             scs0)
            return 0

        lax.fori_loop(0, ND, zissue, 0)
        pltpu.sync_copy(pk_hbm.at[pl.ds(blk0, PKC)], pkbuf)

        def zwait(d, _):
            r0 = row0 + d * DR
            pltpu.make_async_copy(eb0.at[pl.ds(0, DR)],
                                  s_acc.at[pl.ds(r0, DR)], scs0).wait()
            pltpu.make_async_copy(eb0.at[pl.ds(0, DR)],
                                  t_acc.at[pl.ds(r0, DR)], scs0).wait()
            return 0

        lax.fori_loop(0, ND, zwait, 0)
        plsc.subcore_barrier()

        def unpack(i, b):
            r = i % PKC
            for k in range(EB // LANES):
                sl = pl.ds(k * LANES, LANES)
                v = pkbuf[r, sl]
                sv[b][sl] = v & 0xFFFF
                dv[b][sl] = v >> 16

        def issue_loads(i, b):
            pltpu.async_copy(e_hbm.at[c, pl.ds((blk0 + i) * EB, EB)],
                             eb[b], lds[b])
            pltpu.async_copy(xs_hbm.at[c].at[sv[b]], gb[b], lds[b])

        unpack(0, 0)
        issue_loads(0, 0)

        def do_block(i, b):
            nb2 = 1 - b
            pltpu.make_async_copy(
                e_hbm.at[c, pl.ds((blk0 + i) * EB, EB)], eb[b],
                lds[b]).wait()
            pltpu.make_async_copy(
                xs_hbm.at[c].at[sv[b]], gb[b], lds[b]).wait()

            @pl.when(i == PKC - 1)
            def _():
                pltpu.sync_copy(pk_hbm.at[pl.ds(blk0 + PKC, PKC)], pkbuf)

            @pl.when(i + 1 < nblk)
            def _():
                @pl.when(i >= 1)
                def _():
                    pltpu.make_async_copy(
                        eb[nb2], s_acc.at[dv[nb2]], scs[nb2]).wait()
                    pltpu.make_async_copy(
                        gb[nb2], t_acc.at[dv[nb2]], scs[nb2]).wait()

                unpack(i + 1, nb2)
                issue_loads(i + 1, nb2)

            def crow(r, _):
                for k in range(CH):
                    sl = pl.ds(k * LANES, LANES)
                    msg = jnp.maximum(gb[b][r, sl] + eb[b][r, sl],
                                      0.0) + EPS_GEN
                    p = jnp.exp(msg)
                    eb[b][r, sl] = p
                    gb[b][r, sl] = p * msg
                return 0

            lax.fori_loop(0, EB, crow, 0)
            pltpu.async_copy(eb[b], s_acc.at[dv[b]], scs[b], add=True)
            pltpu.async_copy(gb[b], t_acc.at[dv[b]], scs[b], add=True)

        def blockstep(i, _):
            @pl.when(i % 2 == 0)
            def _():
                do_block(i, 0)

            @pl.when(i % 2 == 1)
            def _():
                do_block(i, 1)

            return 0

        lax.fori_loop(0, nblk, blockstep, 0)
        for b in range(2):
            pltpu.make_async_copy(eb[b], s_acc.at[dv[b]], scs[b]).wait()
            pltpu.make_async_copy(gb[b], t_acc.at[dv[b]], scs[b]).wait()
        plsc.subcore_barrier()

        def dissue(d, b):
            r0 = row0 + d * DR
            pltpu.async_copy(s_acc.at[pl.ds(r0, DR)], eb[b].at[pl.ds(0, DR)],
                             lds[b])
            pltpu.async_copy(t_acc.at[pl.ds(r0, DR)], gb[b].at[pl.ds(0, DR)],
                             lds[b])

        dissue(0, 0)

        def dpair(dp, _):
            for b in range(2):
                d = dp * 2 + b
                r0 = row0 + d * DR
                pltpu.make_async_copy(
                    s_acc.at[pl.ds(r0, DR)], eb[b].at[pl.ds(0, DR)],
                    lds[b]).wait()
                pltpu.make_async_copy(
                    t_acc.at[pl.ds(r0, DR)], gb[b].at[pl.ds(0, DR)],
                    lds[b]).wait()

                @pl.when(d + 1 < ND)
                def _():
                    dissue(d + 1, 1 - b)

                def drow(r, _):
                    for k in range(CH):
                        sl = pl.ds(k * LANES, LANES)
                        eb[b][r, sl] = gb[b][r, sl] / (eb[b][r, sl] + 1e-16)
                    return 0

                lax.fori_loop(0, DR, drow, 0)
                pltpu.sync_copy(eb[b].at[pl.ds(0, DR)],
                                out_hbm.at[c, pl.ds(r0, DR)])
            return 0

        lax.fori_loop(0, ND // 2, dpair, 0)

    return aggr


_EB_TC = 2000


def _edge_mlp_body(ea, w1, b1, w2, b2, w3, b3, o1, o2, o3):
    a = ea[...]
    h1 = jnp.dot(a, w1[...], preferred_element_type=jnp.float32) + b1[...]
    h2 = jnp.dot(a, w2[...], preferred_element_type=jnp.float32) + b2[...]
    h3 = jnp.dot(a, w3[...], preferred_element_type=jnp.float32) + b3[...]
    o1[0], o1[1] = h1[:, :64], h1[:, 64:]
    o2[0], o2[1] = h2[:, :32], h2[:, 32:]
    o3[0], o3[1] = h3[:, :64], h3[:, 64:]


def _edge_mlp(edge_attr, p1, p2, p3):
    nblk = N_EDGES // _EB_TC
    wspec = lambda shp: pl.BlockSpec(shp, lambda i: (0, 0))
    ospec = lambda fh: pl.BlockSpec((NC, _EB_TC, fh), lambda i: (0, i, 0))
    return pl.pallas_call(
        _edge_mlp_body,
        grid=(nblk,),
        in_specs=[
            pl.BlockSpec((_EB_TC, D_EDGE), lambda i: (i, 0)),
            wspec((D_EDGE, 128)), wspec((1, 128)),
            wspec((D_EDGE, 64)), wspec((1, 64)),
            wspec((D_EDGE, 128)), wspec((1, 128)),
        ],
        out_specs=[ospec(64), ospec(32), ospec(64)],
        out_shape=[
            jax.ShapeDtypeStruct((NC, N_EDGES, 64), jnp.float32),
            jax.ShapeDtypeStruct((NC, N_EDGES, 32), jnp.float32),
            jax.ShapeDtypeStruct((NC, N_EDGES, 64), jnp.float32),
        ],
    )(edge_attr,
      p1["We"], p1["be"].reshape(1, -1),
      p2["We"], p2["be"].reshape(1, -1),
      p3["We"], p3["be"].reshape(1, -1))


def _bn(h, g, b):
    mu = jnp.mean(h, axis=0, keepdims=True)
    var = jnp.mean((h - mu) ** 2, axis=0, keepdims=True)
    return (h - mu) * lax.rsqrt(var + BN_EPS) * g + b


def _node_body(o_ref, x_ref, wa, ba, gm, bm, wb, bb, gn, bnb, y_ref):
    out = jnp.concatenate([o_ref[0, :N_NODES] + x_ref[0, :N_NODES],
                           o_ref[1, :N_NODES] + x_ref[1, :N_NODES]], axis=1)
    h = jnp.dot(out, wa[...], preferred_element_type=jnp.float32) + ba[...]
    h = jnp.maximum(_bn(h, gm[...], bm[...]), 0.0)
    h = jnp.dot(h, wb[...], preferred_element_type=jnp.float32) + bb[...]
    y = jnp.maximum(_bn(h, gn[...], bnb[...]), 0.0)
    fh = y.shape[1] // 2
    y_ref[0], y_ref[1] = y[:, :fh], y[:, fh:]


def _node_mlp(aggr_out, x_in, p):
    dout = p["Wb"].shape[1]
    return pl.pallas_call(
        _node_body,
        out_shape=jax.ShapeDtypeStruct((NC, N_NODES, dout // 2), jnp.float32),
    )(aggr_out, x_in, p["Wa"], p["ba"].reshape(1, -1), p["gm"].reshape(1, -1),
      p["bm"].reshape(1, -1), p["Wb"], p["bb"].reshape(1, -1),
      p["gn"].reshape(1, -1), p["bn"].reshape(1, -1))


def _node_pool_body(o_ref, x_ref, wa, ba, gm, bm, wb, bb, gn, bnb, batch_ref,
                    out_ref):
    out = jnp.concatenate([o_ref[0, :N_NODES] + x_ref[0, :N_NODES],
                           o_ref[1, :N_NODES] + x_ref[1, :N_NODES]], axis=1)
    h = jnp.dot(out, wa[...], preferred_element_type=jnp.float32) + ba[...]
    h = jnp.maximum(_bn(h, gm[...], bm[...]), 0.0)
    h = jnp.dot(h, wb[...], preferred_element_type=jnp.float32) + bb[...]
    y = jnp.maximum(_bn(h, gn[...], bnb[...]), 0.0)
    onehot = (batch_ref[...] == lax.broadcasted_iota(
        jnp.int32, (1, N_GRAPHS), 1)).astype(jnp.float32)
    s = lax.dot_general(onehot, y, (((0,), (0,)), ((), ())),
                        preferred_element_type=jnp.float32)
    cnt = jnp.sum(onehot, axis=0)
    out_ref[...] = s / jnp.maximum(cnt, 1.0)[:, None]


def _node_pool(aggr_out, x_in, batch2d, p):
    return pl.pallas_call(
        _node_pool_body,
        out_shape=jax.ShapeDtypeStruct((N_GRAPHS, p["Wb"].shape[1]),
                                       jnp.float32),
    )(aggr_out, x_in, p["Wa"], p["ba"].reshape(1, -1), p["gm"].reshape(1, -1),
      p["bm"].reshape(1, -1), p["Wb"], p["bb"].reshape(1, -1),
      p["gn"].reshape(1, -1), p["bn"].reshape(1, -1), batch2d)


def kernel(x, edge_index, edge_attr, batch, params):
    pk = (edge_index[0] | (edge_index[1] << 16)).reshape(NBLK, EB)
    pk = jnp.pad(pk, ((0, PKPAD - NBLK), (0, 0)))
    batch2d = batch.reshape(N_NODES, 1)

    e1, e2, e3 = _edge_mlp(edge_attr, params["l1"], params["l2"], params["l3"])

    sc128 = _make_sc_aggr(128)
    sc64 = _make_sc_aggr(64)
    pad = lambda a: jnp.pad(a, ((0, 0), (0, NP - N_NODES), (0, 0)))
    x0 = pad(x.reshape(N_NODES, NC, 64).transpose(1, 0, 2))
    a1 = sc128(x0, pk, e1)
    x1 = pad(_node_mlp(a1, x0, params["l1"]))
    a2 = sc64(x1, pk, e2)
    x2 = pad(_node_mlp(a2, x1, params["l2"]))
    a3 = sc128(x2, pk, e3)
    return _node_pool(a3, x2, batch2d, params["l3"])
